# baseline (speedup 1.0000x reference)
import functools

import jax
import jax.numpy as jnp
from jax import lax
from jax.experimental import pallas as pl
from jax.experimental.pallas import tpu as pltpu
from jax.experimental.pallas import tpu_sc as plsc

V = 100000
E = 128
B = 1024
H = 50

NC = 2
NS = 16
NW = NC * NS
ROWS_PER_W = B // NW
CHUNK_ROWS = 2
CHUNK_IDX = CHUNK_ROWS * H
NCHUNK = ROWS_PER_W // CHUNK_ROWS
LANES = 16
NREG = E // LANES


def _sc_gather_mean(x_r, emb):
    mesh = plsc.VectorSubcoreMesh(core_axis_name="c", subcore_axis_name="s")

    @functools.partial(
        pl.kernel,
        mesh=mesh,
        out_type=jax.ShapeDtypeStruct((B, E), jnp.float32),
        scratch_types=[
            pltpu.VMEM((NCHUNK, CHUNK_IDX), jnp.int32),
            pltpu.VMEM((CHUNK_IDX, E), jnp.float32),
            pltpu.VMEM((ROWS_PER_W, E), jnp.float32),
            pltpu.SemaphoreType.DMA,
        ],
    )
    def k(x_hbm, emb_hbm, out_hbm, idx_v, buf_v, acc_v, sem):
        wid = lax.axis_index("s") * NC + lax.axis_index("c")
        pltpu.sync_copy(x_hbm.at[wid], idx_v)

        def chunk_body(c, carry):
            pltpu.async_copy(emb_hbm.at[idx_v.at[c]], buf_v, sem).wait()

            def j_body(j, accs):
                return tuple(
                    accs[r * NREG + kk]
                    + buf_v[r * H + j, pl.ds(kk * LANES, LANES)]
                    for r in range(CHUNK_ROWS)
                    for kk in range(NREG)
                )

            init = tuple(
                jnp.zeros((LANES,), jnp.float32)
                for _ in range(CHUNK_ROWS * NREG)
            )
            accs = lax.fori_loop(0, H, j_body, init)
            scale = jnp.float32(1.0 / H)
            for r in range(CHUNK_ROWS):
                for kk in range(NREG):
                    acc_v[c * CHUNK_ROWS + r, pl.ds(kk * LANES, LANES)] = (
                        accs[r * NREG + kk] * scale
                    )
            return carry

        lax.fori_loop(0, NCHUNK, chunk_body, 0)
        pltpu.sync_copy(acc_v, out_hbm.at[pl.ds(wid * ROWS_PER_W, ROWS_PER_W)])

    return k(x_r, emb)


VT = 2048
NV = (V + VT - 1) // VT
TAIL = V - (NV - 1) * VT
NBUF = 3


def _pa_body(avgb_ref, w_ref, b_ref, m_ref, macc):
    j = pl.program_id(0)
    l32 = jnp.dot(avgb_ref[...], w_ref[...],
                  preferred_element_type=jnp.float32)
    l = (l32 + b_ref[...]).astype(jnp.bfloat16)

    @pl.when(j == 0)
    def _():
        macc[...] = l

    @pl.when((j > 0) & (j < NV - 1))
    def _():
        macc[...] = jnp.maximum(macc[...], l)

    @pl.when(j == NV - 1)
    def _():
        col = lax.broadcasted_iota(jnp.int32, (1, VT), 1)
        lm = jnp.where(col < TAIL, l, jnp.finfo(jnp.bfloat16).min)
        macc[...] = jnp.maximum(macc[...], lm)
        m_ref[...] = jnp.max(macc[...], axis=1, keepdims=True).astype(
            jnp.float32)


def _pb_body(avgb_ref, w_ref, b_ref, m_ref, s_ref, sacc):
    j = pl.program_id(0)
    l = jnp.dot(avgb_ref[...], w_ref[...],
                preferred_element_type=jnp.float32)
    e = jnp.exp(l + b_ref[...] - m_ref[...])

    @pl.when(j == 0)
    def _():
        sacc[...] = e

    @pl.when((j > 0) & (j < NV - 1))
    def _():
        sacc[...] = sacc[...] + e

    @pl.when(j == NV - 1)
    def _():
        col = lax.broadcasted_iota(jnp.int32, (1, VT), 1)
        sacc[...] = sacc[...] + jnp.where(col < TAIL, e, 0.0)
        s_ref[...] = jnp.sum(sacc[...], axis=1, keepdims=True)


def _pc_body(avgb_ref, w_ref, b_ref, c_ref, out_hbm, buf0, buf1, buf2,
             sems):
    j = pl.program_id(0)
    jm = lax.rem(j, NBUF)
    bufs = [buf0, buf1, buf2]

    l = jnp.dot(avgb_ref[...], w_ref[...],
                preferred_element_type=jnp.float32)
    e = jnp.exp(l + b_ref[...] - c_ref[...])

    for bsel in range(NBUF):
        @pl.when((jm == bsel) & (j >= NBUF))
        def _(bsel=bsel):
            off = pl.multiple_of((j - NBUF) * VT, VT)
            pltpu.make_async_copy(
                bufs[bsel],
                out_hbm.at[:, pl.ds(off, VT)],
                sems.at[bsel],
            ).wait()

    for bsel in range(NBUF):
        @pl.when(jm == bsel)
        def _(bsel=bsel):
            bufs[bsel][...] = e
            off = pl.multiple_of(j * VT, VT)
            pltpu.make_async_copy(
                bufs[bsel],
                out_hbm.at[:, pl.ds(off, VT)],
                sems.at[bsel],
            ).start()

    @pl.when(j == NV - 2)
    def _():
        for k in range(NBUF):
            jj = NV - 2 - k
            pltpu.make_async_copy(
                bufs[jj % NBUF],
                out_hbm.at[:, pl.ds(jj * VT, VT)],
                sems.at[jj % NBUF],
            ).wait()


def _pc_tail_body(avgb_ref, w_ref, b_ref, c_ref, prev_ref, out_ref):
    del prev_ref
    l = jnp.dot(avgb_ref[...], w_ref[...],
                preferred_element_type=jnp.float32)
    out_ref[...] = jnp.exp(l + b_ref[...] - c_ref[...])


def _tc_softmax(avgb, Wb, b2, b16, interpret=False):
    m = pl.pallas_call(
        _pa_body,
        grid=(NV,),
        in_specs=[
            pl.BlockSpec((B, E), lambda j: (0, 0)),
            pl.BlockSpec((E, VT), lambda j: (0, j)),
            pl.BlockSpec((1, VT), lambda j: (0, j)),
        ],
        out_specs=pl.BlockSpec((B, 1), lambda j: (0, 0)),
        out_shape=jax.ShapeDtypeStruct((B, 1), jnp.float32),
        scratch_shapes=[pltpu.VMEM((B, VT), jnp.bfloat16)],
        interpret=interpret,
    )(avgb, Wb, b2)

    s = pl.pallas_call(
        _pb_body,
        grid=(NV,),
        in_specs=[
            pl.BlockSpec((B, E), lambda j: (0, 0)),
            pl.BlockSpec((E, VT), lambda j: (0, j)),
            pl.BlockSpec((1, VT), lambda j: (0, j)),
            pl.BlockSpec((B, 1), lambda j: (0, 0)),
        ],
        out_specs=pl.BlockSpec((B, 1), lambda j: (0, 0)),
        out_shape=jax.ShapeDtypeStruct((B, 1), jnp.float32),
        scratch_shapes=[pltpu.VMEM((B, VT), jnp.float32)],
        interpret=interpret,
    )(avgb, Wb, b2, m)

    c = m + jnp.log(s)

    out = pl.pallas_call(
        _pc_body,
        grid=(NV - 1,),
        in_specs=[
            pl.BlockSpec((B, E), lambda j: (0, 0)),
            pl.BlockSpec((E, VT), lambda j: (0, j)),
            pl.BlockSpec((1, VT), lambda j: (0, j)),
            pl.BlockSpec((B, 1), lambda j: (0, 0)),
        ],
        out_specs=pl.BlockSpec(memory_space=pl.ANY),
        out_shape=jax.ShapeDtypeStruct((B, V), jnp.float32),
        scratch_shapes=[
            pltpu.VMEM((B, VT), jnp.float32),
            pltpu.VMEM((B, VT), jnp.float32),
            pltpu.VMEM((B, VT), jnp.float32),
            pltpu.SemaphoreType.DMA((NBUF,)),
        ],
        compiler_params=pltpu.CompilerParams(
            dimension_semantics=("arbitrary",),
        ),
        interpret=interpret,
    )(avgb, Wb, b2, c)

    out = pl.pallas_call(
        _pc_tail_body,
        grid=(1,),
        in_specs=[
            pl.BlockSpec((B, E), lambda j: (0, 0)),
            pl.BlockSpec((E, VT), lambda j: (0, NV - 1)),
            pl.BlockSpec((1, VT), lambda j: (0, NV - 1)),
            pl.BlockSpec((B, 1), lambda j: (0, 0)),
            pl.BlockSpec(memory_space=pl.ANY),
        ],
        out_specs=pl.BlockSpec((B, VT), lambda j: (0, NV - 1)),
        out_shape=jax.ShapeDtypeStruct((B, V), jnp.float32),
        input_output_aliases={4: 0},
        interpret=interpret,
    )(avgb, Wb, b2, c, out)
    return out


def kernel(x, emb, W, b):
    x_r = x.astype(jnp.int32).reshape(NW, NCHUNK, CHUNK_IDX)
    avg = _sc_gather_mean(x_r, emb)
    avgb = avg.astype(jnp.bfloat16)
    Wb = W.astype(jnp.bfloat16)
    b2 = b.reshape(1, V)
    b16 = b2.astype(jnp.bfloat16)
    return _tc_softmax(avgb, Wb, b2, b16)


# speedup vs baseline: 1.0014x; 1.0014x over previous
import functools

import jax
import jax.numpy as jnp
from jax import lax
from jax.experimental import pallas as pl
from jax.experimental.pallas import tpu as pltpu
from jax.experimental.pallas import tpu_sc as plsc

V = 100000
E = 128
B = 1024
H = 50

NC = 2
NS = 16
NW = NC * NS
ROWS_PER_W = B // NW
CHUNK_ROWS = 2
CHUNK_IDX = CHUNK_ROWS * H
NCHUNK = ROWS_PER_W // CHUNK_ROWS
LANES = 16
NREG = E // LANES


def _sc_gather_mean(x_r, emb):
    mesh = plsc.VectorSubcoreMesh(core_axis_name="c", subcore_axis_name="s")

    @functools.partial(
        pl.kernel,
        mesh=mesh,
        out_type=jax.ShapeDtypeStruct((B, E), jnp.float32),
        scratch_types=[
            pltpu.VMEM((NCHUNK, CHUNK_IDX), jnp.int32),
            pltpu.VMEM((CHUNK_IDX, E), jnp.float32),
            pltpu.VMEM((ROWS_PER_W, E), jnp.float32),
            pltpu.SemaphoreType.DMA,
        ],
    )
    def k(x_hbm, emb_hbm, out_hbm, idx_v, buf_v, acc_v, sem):
        wid = lax.axis_index("s") * NC + lax.axis_index("c")
        pltpu.sync_copy(x_hbm.at[wid], idx_v)

        def chunk_body(c, carry):
            pltpu.async_copy(emb_hbm.at[idx_v.at[c]], buf_v, sem).wait()

            def j_body(j, accs):
                return tuple(
                    accs[r * NREG + kk]
                    + buf_v[r * H + j, pl.ds(kk * LANES, LANES)]
                    for r in range(CHUNK_ROWS)
                    for kk in range(NREG)
                )

            init = tuple(
                jnp.zeros((LANES,), jnp.float32)
                for _ in range(CHUNK_ROWS * NREG)
            )
            accs = lax.fori_loop(0, H, j_body, init)
            scale = jnp.float32(1.0 / H)
            for r in range(CHUNK_ROWS):
                for kk in range(NREG):
                    acc_v[c * CHUNK_ROWS + r, pl.ds(kk * LANES, LANES)] = (
                        accs[r * NREG + kk] * scale
                    )
            return carry

        lax.fori_loop(0, NCHUNK, chunk_body, 0)
        pltpu.sync_copy(acc_v, out_hbm.at[pl.ds(wid * ROWS_PER_W, ROWS_PER_W)])

    return k(x_r, emb)


VT = 2048
NV = (V + VT - 1) // VT
TAIL = V - (NV - 1) * VT
NBUF = 4


def _pa_body(avgb_ref, w_ref, b_ref, m_ref, macc):
    j = pl.program_id(0)
    l32 = jnp.dot(avgb_ref[...], w_ref[...],
                  preferred_element_type=jnp.float32)
    l = (l32 + b_ref[...]).astype(jnp.bfloat16)

    @pl.when(j == 0)
    def _():
        macc[...] = l

    @pl.when((j > 0) & (j < NV - 1))
    def _():
        macc[...] = jnp.maximum(macc[...], l)

    @pl.when(j == NV - 1)
    def _():
        col = lax.broadcasted_iota(jnp.int32, (1, VT), 1)
        lm = jnp.where(col < TAIL, l, jnp.finfo(jnp.bfloat16).min)
        macc[...] = jnp.maximum(macc[...], lm)
        m_ref[...] = jnp.max(macc[...], axis=1, keepdims=True).astype(
            jnp.float32)


def _pb_body(avgb_ref, w_ref, b_ref, m_ref, s_ref, sacc):
    j = pl.program_id(0)
    l = jnp.dot(avgb_ref[...], w_ref[...],
                preferred_element_type=jnp.float32)
    e = jnp.exp(l + b_ref[...] - m_ref[...])

    @pl.when(j == 0)
    def _():
        sacc[...] = e

    @pl.when((j > 0) & (j < NV - 1))
    def _():
        sacc[...] = sacc[...] + e

    @pl.when(j == NV - 1)
    def _():
        col = lax.broadcasted_iota(jnp.int32, (1, VT), 1)
        sacc[...] = sacc[...] + jnp.where(col < TAIL, e, 0.0)
        s_ref[...] = jnp.sum(sacc[...], axis=1, keepdims=True)


def _pc_body(avgb_ref, w_ref, b_ref, c_ref, out_hbm, bufs, sems):
    j = pl.program_id(0)
    jm = lax.rem(j, NBUF)

    @pl.when(j >= NBUF)
    def _():
        off = pl.multiple_of((j - NBUF) * VT, VT)
        pltpu.make_async_copy(
            bufs.at[jm],
            out_hbm.at[:, pl.ds(off, VT)],
            sems.at[jm],
        ).wait()

    l = jnp.dot(avgb_ref[...], w_ref[...],
                preferred_element_type=jnp.float32)
    bufs[jm] = jnp.exp(l + b_ref[...] - c_ref[...])
    off = pl.multiple_of(j * VT, VT)
    for pri in range(2):
        @pl.when(lax.rem(j, 2) == pri)
        def _(pri=pri):
            pltpu.make_async_copy(
                bufs.at[jm],
                out_hbm.at[:, pl.ds(off, VT)],
                sems.at[jm],
            ).start(priority=pri)

    @pl.when(j == NV - 2)
    def _():
        for k in range(NBUF):
            jj = NV - 2 - k
            pltpu.make_async_copy(
                bufs.at[jj % NBUF],
                out_hbm.at[:, pl.ds(jj * VT, VT)],
                sems.at[jj % NBUF],
            ).wait()


def _pc_tail_body(avgb_ref, w_ref, b_ref, c_ref, prev_ref, out_ref):
    del prev_ref
    l = jnp.dot(avgb_ref[...], w_ref[...],
                preferred_element_type=jnp.float32)
    out_ref[...] = jnp.exp(l + b_ref[...] - c_ref[...])


def _tc_softmax(avgb, Wb, b2, b16, interpret=False):
    m = pl.pallas_call(
        _pa_body,
        grid=(NV,),
        in_specs=[
            pl.BlockSpec((B, E), lambda j: (0, 0)),
            pl.BlockSpec((E, VT), lambda j: (0, j)),
            pl.BlockSpec((1, VT), lambda j: (0, j)),
        ],
        out_specs=pl.BlockSpec((B, 1), lambda j: (0, 0)),
        out_shape=jax.ShapeDtypeStruct((B, 1), jnp.float32),
        scratch_shapes=[pltpu.VMEM((B, VT), jnp.bfloat16)],
        interpret=interpret,
    )(avgb, Wb, b2)

    s = pl.pallas_call(
        _pb_body,
        grid=(NV,),
        in_specs=[
            pl.BlockSpec((B, E), lambda j: (0, 0)),
            pl.BlockSpec((E, VT), lambda j: (0, j)),
            pl.BlockSpec((1, VT), lambda j: (0, j)),
            pl.BlockSpec((B, 1), lambda j: (0, 0)),
        ],
        out_specs=pl.BlockSpec((B, 1), lambda j: (0, 0)),
        out_shape=jax.ShapeDtypeStruct((B, 1), jnp.float32),
        scratch_shapes=[pltpu.VMEM((B, VT), jnp.float32)],
        interpret=interpret,
    )(avgb, Wb, b2, m)

    c = m + jnp.log(s)

    out = pl.pallas_call(
        _pc_body,
        grid=(NV - 1,),
        in_specs=[
            pl.BlockSpec((B, E), lambda j: (0, 0)),
            pl.BlockSpec((E, VT), lambda j: (0, j)),
            pl.BlockSpec((1, VT), lambda j: (0, j)),
            pl.BlockSpec((B, 1), lambda j: (0, 0)),
        ],
        out_specs=pl.BlockSpec(memory_space=pl.ANY),
        out_shape=jax.ShapeDtypeStruct((B, V), jnp.float32),
        scratch_shapes=[
            pltpu.VMEM((NBUF, B, VT), jnp.float32),
            pltpu.SemaphoreType.DMA((NBUF,)),
        ],
        compiler_params=pltpu.CompilerParams(
            dimension_semantics=("arbitrary",),
        ),
        interpret=interpret,
    )(avgb, Wb, b2, c)

    out = pl.pallas_call(
        _pc_tail_body,
        grid=(1,),
        in_specs=[
            pl.BlockSpec((B, E), lambda j: (0, 0)),
            pl.BlockSpec((E, VT), lambda j: (0, NV - 1)),
            pl.BlockSpec((1, VT), lambda j: (0, NV - 1)),
            pl.BlockSpec((B, 1), lambda j: (0, 0)),
            pl.BlockSpec(memory_space=pl.ANY),
        ],
        out_specs=pl.BlockSpec((B, VT), lambda j: (0, NV - 1)),
        out_shape=jax.ShapeDtypeStruct((B, V), jnp.float32),
        input_output_aliases={4: 0},
        interpret=interpret,
    )(avgb, Wb, b2, c, out)
    return out


def kernel(x, emb, W, b):
    x_r = x.astype(jnp.int32).reshape(NW, NCHUNK, CHUNK_IDX)
    avg = _sc_gather_mean(x_r, emb)
    avgb = avg.astype(jnp.bfloat16)
    Wb = W.astype(jnp.bfloat16)
    b2 = b.reshape(1, V)
    b16 = b2.astype(jnp.bfloat16)
    return _tc_softmax(avgb, Wb, b2, b16)
